# baseline (speedup 1.0000x reference)
import functools

import jax
import jax.numpy as jnp
from jax import lax
from jax.experimental import pallas as pl
from jax.experimental.pallas import tpu as pltpu
from jax.experimental.pallas import tpu_sc as plsc

N_NODES = 10000
N_EDGES = 320000
D_FEAT = 128
DW = D_FEAT // 2

NC = 2
NS = 16
LANES = 16
NW = NC * NS
CHUNK = 80
GROUPS = CHUNK // LANES
QUAD = 4
SB = 16
NSUPER = 8
EPW = NSUPER * SB * CHUNK
E_PAD = NW * EPW
RPT = 624
RTAIL = N_NODES - NS * RPT

MASK_HI = -65536


def _scale_chunk(gbuf, sbuf, val_v, c):
    def group_body(g, carry):
        v16 = val_v[c, pl.ds(g * LANES, LANES)]
        for i in range(LANES):
            s = v16.at[jnp.full((LANES,), i, jnp.int32)].get(
                mode="promise_in_bounds")
            e = g * LANES + i
            for k in range(4):
                w = gbuf[e, pl.ds(k * LANES, LANES)]
                lo = jax.lax.bitcast_convert_type(w << jnp.full((LANES,), 16, jnp.int32), jnp.float32)
                hi = jax.lax.bitcast_convert_type(w & jnp.full((LANES,), MASK_HI, jnp.int32), jnp.float32)
                sbuf[e, pl.ds(32 * k, LANES)] = lo * s
                sbuf[e, pl.ds(32 * k + LANES, LANES)] = hi * s
        return carry

    lax.fori_loop(0, GROUPS, group_body, 0)


def _sc_body(x_hbm, row_hbm, col_hbm, val_hbm, z_hbm, out_hbm,
             row_v, col_v, val_v, gb0, gb1, gb2, gb3, sb0, sb1, acc_sh,
             gsem0, gsem1, gsem2, gsem3, ssem0, ssem1):
    cid = lax.axis_index("c")
    sid = lax.axis_index("s")
    wid = sid * NC + cid
    gbufs = (gb0, gb1, gb2, gb3)
    gsems = (gsem0, gsem1, gsem2, gsem3)
    sbufs = (sb0, sb1)
    ssems = (ssem0, ssem1)

    pltpu.sync_copy(z_hbm.at[pl.ds(sid * RPT, RPT)],
                    acc_sh.at[pl.ds(sid * RPT, RPT)])

    @pl.when(sid == 0)
    def _():
        pltpu.sync_copy(z_hbm.at[pl.ds(NS * RPT, RTAIL)],
                        acc_sh.at[pl.ds(NS * RPT, RTAIL)])

    plsc.subcore_barrier()

    def super_body(s_i, carry0):
        pltpu.sync_copy(row_hbm.at[wid, s_i], row_v)
        pltpu.sync_copy(col_hbm.at[wid, s_i], col_v)
        pltpu.sync_copy(val_hbm.at[wid, s_i], val_v)

        def quad_body(p, carry):
            gathers = []
            for b in range(QUAD):
                c = p * QUAD + b
                gathers.append(pltpu.async_copy(
                    x_hbm.at[col_v.at[c]], gbufs[b], gsems[b]))
            scatters = [None, None]
            for b in range(QUAD):
                c = p * QUAD + b
                gathers[b].wait()
                if b >= 2:
                    scatters[b % 2].wait()
                _scale_chunk(gbufs[b], sbufs[b % 2], val_v, c)
                scatters[b % 2] = pltpu.async_copy(
                    sbufs[b % 2], acc_sh.at[row_v.at[c]], ssems[b % 2],
                    add=True)
            scatters[0].wait()
            scatters[1].wait()
            return carry

        lax.fori_loop(0, SB // QUAD, quad_body, 0)
        return carry0

    lax.fori_loop(0, NSUPER, super_body, 0)
    plsc.subcore_barrier()

    pltpu.sync_copy(acc_sh.at[pl.ds(sid * RPT, RPT)],
                    out_hbm.at[cid, pl.ds(sid * RPT, RPT)])

    @pl.when(sid == 0)
    def _():
        pltpu.sync_copy(acc_sh.at[pl.ds(NS * RPT, RTAIL)],
                        out_hbm.at[cid, pl.ds(NS * RPT, RTAIL)])


def _combine_body(p_ref, o_ref):
    o_ref[...] = p_ref[0] + p_ref[1]


@jax.jit
def kernel(x, adj_indices, adj_values):
    idx = adj_indices.astype(jnp.int32)
    pad = E_PAD - N_EDGES
    spread = (jnp.arange(pad, dtype=jnp.int32) * 13) % N_NODES
    row = jnp.concatenate([idx[0], spread])
    col = jnp.concatenate([idx[1], spread])
    val = jnp.concatenate([adj_values, jnp.zeros((pad,), jnp.float32)])
    row4 = row.reshape(NW, NSUPER, SB, CHUNK)
    col4 = col.reshape(NW, NSUPER, SB, CHUNK)
    val4 = val.reshape(NW, NSUPER, SB, CHUNK)
    zeros = jnp.zeros((N_NODES, D_FEAT), jnp.float32)

    x_bf = x.astype(jnp.bfloat16).reshape(N_NODES, 4, 2, LANES)
    x_il = jnp.swapaxes(x_bf, 2, 3).reshape(N_NODES, DW, 2)
    x_pk = jax.lax.bitcast_convert_type(x_il, jnp.int32)

    mesh = plsc.VectorSubcoreMesh(core_axis_name="c", subcore_axis_name="s",
                                  num_cores=NC, num_subcores=NS)
    partials = pl.kernel(
        _sc_body,
        out_type=jax.ShapeDtypeStruct((NC, N_NODES, D_FEAT), jnp.float32),
        mesh=mesh,
        compiler_params=pltpu.CompilerParams(use_tc_tiling_on_sc=False),
        scratch_types=[
            pltpu.VMEM((SB, CHUNK), jnp.int32),
            pltpu.VMEM((SB, CHUNK), jnp.int32),
            pltpu.VMEM((SB, CHUNK), jnp.float32),
            pltpu.VMEM((CHUNK, DW), jnp.int32),
            pltpu.VMEM((CHUNK, DW), jnp.int32),
            pltpu.VMEM((CHUNK, DW), jnp.int32),
            pltpu.VMEM((CHUNK, DW), jnp.int32),
            pltpu.VMEM((CHUNK, D_FEAT), jnp.float32),
            pltpu.VMEM((CHUNK, D_FEAT), jnp.float32),
            pltpu.VMEM_SHARED((N_NODES, D_FEAT), jnp.float32),
            pltpu.SemaphoreType.DMA,
            pltpu.SemaphoreType.DMA,
            pltpu.SemaphoreType.DMA,
            pltpu.SemaphoreType.DMA,
            pltpu.SemaphoreType.DMA,
            pltpu.SemaphoreType.DMA,
        ],
    )(x_pk, row4, col4, val4, zeros)

    blk = 1000
    return pl.pallas_call(
        _combine_body,
        out_shape=jax.ShapeDtypeStruct((N_NODES, D_FEAT), jnp.float32),
        grid=(N_NODES // blk,),
        in_specs=[pl.BlockSpec((NC, blk, D_FEAT), lambda i: (0, i, 0))],
        out_specs=pl.BlockSpec((blk, D_FEAT), lambda i: (i, 0)),
    )(partials)


# speedup vs baseline: 1.6495x; 1.6495x over previous
import functools

import jax
import jax.numpy as jnp
from jax import lax
from jax.experimental import pallas as pl
from jax.experimental.pallas import tpu as pltpu
from jax.experimental.pallas import tpu_sc as plsc

N_NODES = 10000
N_EDGES = 320000
D_FEAT = 128

NC = 2
NS = 16
LANES = 16
NW = NC * NS
CHUNK = 80
GROUPS = CHUNK // LANES
QUAD = 4
SB = 16
NSUPER = 8
EPW = NSUPER * SB * CHUNK
E_PAD = NW * EPW
RPT = 624
RTAIL = N_NODES - NS * RPT


def _scale_chunk(gbuf, val_v, c):
    def group_body(g, carry):
        v16 = val_v[c, pl.ds(g * LANES, LANES)]
        for i in range(LANES):
            s = v16.at[jnp.full((LANES,), i, jnp.int32)].get(
                mode="promise_in_bounds")
            e = g * LANES + i
            for f in range(D_FEAT // LANES):
                sl = pl.ds(f * LANES, LANES)
                gbuf[e, sl] = gbuf[e, sl] * s
        return carry

    lax.fori_loop(0, GROUPS, group_body, 0)


def _sc_body(x_hbm, row_hbm, col_hbm, val_hbm, z_hbm, out_hbm,
             row_v, col_v, val_v, gb0, gb1, gb2, gb3, acc_sh,
             gsem0, gsem1, gsem2, gsem3, ssem0, ssem1, ssem2, ssem3):
    gbufs = (gb0, gb1, gb2, gb3)
    gsems = (gsem0, gsem1, gsem2, gsem3)
    ssems = (ssem0, ssem1, ssem2, ssem3)
    cid = lax.axis_index("c")
    sid = lax.axis_index("s")
    wid = sid * NC + cid

    pltpu.sync_copy(z_hbm.at[pl.ds(sid * RPT, RPT)],
                    acc_sh.at[pl.ds(sid * RPT, RPT)])

    @pl.when(sid == 0)
    def _():
        pltpu.sync_copy(z_hbm.at[pl.ds(NS * RPT, RTAIL)],
                        acc_sh.at[pl.ds(NS * RPT, RTAIL)])

    plsc.subcore_barrier()

    def super_body(s_i, carry0):
        pltpu.sync_copy(row_hbm.at[wid, s_i], row_v)
        pltpu.sync_copy(col_hbm.at[wid, s_i], col_v)
        pltpu.sync_copy(val_hbm.at[wid, s_i], val_v)

        def quad_body(p, carry):
            gathers = []
            for b in range(QUAD):
                c = p * QUAD + b
                h = CHUNK // 2
                gathers.append((
                    pltpu.async_copy(x_hbm.at[col_v.at[c, pl.ds(0, h)]],
                                     gbufs[b].at[pl.ds(0, h)], gsems[b]),
                    pltpu.async_copy(x_hbm.at[col_v.at[c, pl.ds(h, h)]],
                                     gbufs[b].at[pl.ds(h, h)], gsems[b])))
            scatters = []
            for b in range(QUAD):
                c = p * QUAD + b
                gathers[b][0].wait()
                gathers[b][1].wait()
                _scale_chunk(gbufs[b], val_v, c)
                scatters.append(pltpu.async_copy(
                    gbufs[b], acc_sh.at[row_v.at[c]], ssems[b], add=True))
            for b in range(QUAD):
                scatters[b].wait()
            return carry

        lax.fori_loop(0, SB // QUAD, quad_body, 0)
        return carry0

    lax.fori_loop(0, NSUPER, super_body, 0)
    plsc.subcore_barrier()

    pltpu.sync_copy(acc_sh.at[pl.ds(sid * RPT, RPT)],
                    out_hbm.at[cid, pl.ds(sid * RPT, RPT)])

    @pl.when(sid == 0)
    def _():
        pltpu.sync_copy(acc_sh.at[pl.ds(NS * RPT, RTAIL)],
                        out_hbm.at[cid, pl.ds(NS * RPT, RTAIL)])


def _combine_body(p_ref, o_ref):
    o_ref[...] = p_ref[0] + p_ref[1]


@jax.jit
def kernel(x, adj_indices, adj_values):
    idx = adj_indices.astype(jnp.int32)
    pad = E_PAD - N_EDGES
    spread = (jnp.arange(pad, dtype=jnp.int32) * 13) % N_NODES
    row = jnp.concatenate([idx[0], spread])
    col = jnp.concatenate([idx[1], spread])
    val = jnp.concatenate([adj_values, jnp.zeros((pad,), jnp.float32)])
    row4 = row.reshape(NW, NSUPER, SB, CHUNK)
    col4 = col.reshape(NW, NSUPER, SB, CHUNK)
    val4 = val.reshape(NW, NSUPER, SB, CHUNK)
    zeros = jnp.zeros((N_NODES, D_FEAT), jnp.float32)

    mesh = plsc.VectorSubcoreMesh(core_axis_name="c", subcore_axis_name="s",
                                  num_cores=NC, num_subcores=NS)
    partials = pl.kernel(
        _sc_body,
        out_type=jax.ShapeDtypeStruct((NC, N_NODES, D_FEAT), jnp.float32),
        mesh=mesh,
        scratch_types=[
            pltpu.VMEM((SB, CHUNK), jnp.int32),
            pltpu.VMEM((SB, CHUNK), jnp.int32),
            pltpu.VMEM((SB, CHUNK), jnp.float32),
            pltpu.VMEM((CHUNK, D_FEAT), jnp.float32),
            pltpu.VMEM((CHUNK, D_FEAT), jnp.float32),
            pltpu.VMEM((CHUNK, D_FEAT), jnp.float32),
            pltpu.VMEM((CHUNK, D_FEAT), jnp.float32),
            pltpu.VMEM_SHARED((N_NODES, D_FEAT), jnp.float32),
            pltpu.SemaphoreType.DMA,
            pltpu.SemaphoreType.DMA,
            pltpu.SemaphoreType.DMA,
            pltpu.SemaphoreType.DMA,
            pltpu.SemaphoreType.DMA,
            pltpu.SemaphoreType.DMA,
            pltpu.SemaphoreType.DMA,
            pltpu.SemaphoreType.DMA,
        ],
    )(x, row4, col4, val4, zeros)

    blk = 1000
    return pl.pallas_call(
        _combine_body,
        out_shape=jax.ShapeDtypeStruct((N_NODES, D_FEAT), jnp.float32),
        grid=(N_NODES // blk,),
        in_specs=[pl.BlockSpec((NC, blk, D_FEAT), lambda i: (0, i, 0))],
        out_specs=pl.BlockSpec((blk, D_FEAT), lambda i: (i, 0)),
    )(partials)


# speedup vs baseline: 1.7126x; 1.0383x over previous
import functools

import jax
import jax.numpy as jnp
from jax import lax
from jax.experimental import pallas as pl
from jax.experimental.pallas import tpu as pltpu
from jax.experimental.pallas import tpu_sc as plsc

N_NODES = 10000
N_EDGES = 320000
D_FEAT = 128

NC = 2
NS = 16
LANES = 16
NW = NC * NS
CHUNK = 80
GROUPS = CHUNK // LANES
QUAD = 4
SB = 16
NSUPER = 8
EPW = NSUPER * SB * CHUNK
E_PAD = NW * EPW
RPT = 624
RTAIL = N_NODES - NS * RPT


def _scale_chunk(gbuf, val_v, c):
    def group_body(g, carry):
        v16 = val_v[c, pl.ds(g * LANES, LANES)]
        for i in range(LANES):
            s = v16.at[jnp.full((LANES,), i, jnp.int32)].get(
                mode="promise_in_bounds")
            e = g * LANES + i
            for f in range(D_FEAT // LANES):
                sl = pl.ds(f * LANES, LANES)
                gbuf[e, sl] = gbuf[e, sl] * s
        return carry

    lax.fori_loop(0, GROUPS, group_body, 0)


def _sc_body(x_hbm, row_hbm, col_hbm, val_hbm, out_hbm,
             row_v, col_v, val_v, gb0, gb1, gb2, gb3, acc_sh,
             gsem0, gsem1, gsem2, gsem3, ssem0, ssem1, ssem2, ssem3):
    gbufs = (gb0, gb1, gb2, gb3)
    gsems = (gsem0, gsem1, gsem2, gsem3)
    ssems = (ssem0, ssem1, ssem2, ssem3)
    cid = lax.axis_index("c")
    sid = lax.axis_index("s")
    wid = sid * NC + cid

    def zfill(r, carry):
        for f in range(D_FEAT // LANES):
            gb0[r, pl.ds(f * LANES, LANES)] = jnp.zeros((LANES,), jnp.float32)
        return carry

    lax.fori_loop(0, CHUNK, zfill, 0)
    ZROWS = 78
    for z in range(8):
        pltpu.sync_copy(gb0.at[pl.ds(0, ZROWS)],
                        acc_sh.at[pl.ds(sid * RPT + z * ZROWS, ZROWS)])

    @pl.when(sid == 0)
    def _():
        pltpu.sync_copy(gb0.at[pl.ds(0, RTAIL)],
                        acc_sh.at[pl.ds(NS * RPT, RTAIL)])

    plsc.subcore_barrier()

    def super_body(s_i, carry0):
        pltpu.sync_copy(row_hbm.at[wid, s_i], row_v)
        pltpu.sync_copy(col_hbm.at[wid, s_i], col_v)
        pltpu.sync_copy(val_hbm.at[wid, s_i], val_v)

        def quad_body(p, carry):
            gathers = []
            for b in range(QUAD):
                c = p * QUAD + b
                h = CHUNK // 2
                gathers.append((
                    pltpu.async_copy(x_hbm.at[col_v.at[c, pl.ds(0, h)]],
                                     gbufs[b].at[pl.ds(0, h)], gsems[b]),
                    pltpu.async_copy(x_hbm.at[col_v.at[c, pl.ds(h, h)]],
                                     gbufs[b].at[pl.ds(h, h)], gsems[b])))
            scatters = []
            for b in range(QUAD):
                c = p * QUAD + b
                gathers[b][0].wait()
                gathers[b][1].wait()
                _scale_chunk(gbufs[b], val_v, c)
                scatters.append(pltpu.async_copy(
                    gbufs[b], acc_sh.at[row_v.at[c]], ssems[b], add=True))
            for b in range(QUAD):
                scatters[b].wait()
            return carry

        lax.fori_loop(0, SB // QUAD, quad_body, 0)
        return carry0

    lax.fori_loop(0, NSUPER, super_body, 0)
    plsc.subcore_barrier()

    pltpu.sync_copy(acc_sh.at[pl.ds(sid * RPT, RPT)],
                    out_hbm.at[cid, pl.ds(sid * RPT, RPT)])

    @pl.when(sid == 0)
    def _():
        pltpu.sync_copy(acc_sh.at[pl.ds(NS * RPT, RTAIL)],
                        out_hbm.at[cid, pl.ds(NS * RPT, RTAIL)])


def _combine_body(p_ref, o_ref):
    o_ref[...] = p_ref[0] + p_ref[1]


@jax.jit
def kernel(x, adj_indices, adj_values):
    idx = adj_indices.astype(jnp.int32)
    pad = E_PAD - N_EDGES
    spread = (jnp.arange(pad, dtype=jnp.int32) * 13) % N_NODES
    row = jnp.concatenate([idx[0], spread])
    col = jnp.concatenate([idx[1], spread])
    val = jnp.concatenate([adj_values, jnp.zeros((pad,), jnp.float32)])
    row4 = row.reshape(NW, NSUPER, SB, CHUNK)
    col4 = col.reshape(NW, NSUPER, SB, CHUNK)
    val4 = val.reshape(NW, NSUPER, SB, CHUNK)

    mesh = plsc.VectorSubcoreMesh(core_axis_name="c", subcore_axis_name="s",
                                  num_cores=NC, num_subcores=NS)
    partials = pl.kernel(
        _sc_body,
        out_type=jax.ShapeDtypeStruct((NC, N_NODES, D_FEAT), jnp.float32),
        mesh=mesh,
        scratch_types=[
            pltpu.VMEM((SB, CHUNK), jnp.int32),
            pltpu.VMEM((SB, CHUNK), jnp.int32),
            pltpu.VMEM((SB, CHUNK), jnp.float32),
            pltpu.VMEM((CHUNK, D_FEAT), jnp.float32),
            pltpu.VMEM((CHUNK, D_FEAT), jnp.float32),
            pltpu.VMEM((CHUNK, D_FEAT), jnp.float32),
            pltpu.VMEM((CHUNK, D_FEAT), jnp.float32),
            pltpu.VMEM_SHARED((N_NODES, D_FEAT), jnp.float32),
            pltpu.SemaphoreType.DMA,
            pltpu.SemaphoreType.DMA,
            pltpu.SemaphoreType.DMA,
            pltpu.SemaphoreType.DMA,
            pltpu.SemaphoreType.DMA,
            pltpu.SemaphoreType.DMA,
            pltpu.SemaphoreType.DMA,
            pltpu.SemaphoreType.DMA,
        ],
    )(x, row4, col4, val4)

    blk = 2000
    return pl.pallas_call(
        _combine_body,
        out_shape=jax.ShapeDtypeStruct((N_NODES, D_FEAT), jnp.float32),
        grid=(N_NODES // blk,),
        in_specs=[pl.BlockSpec((NC, blk, D_FEAT), lambda i: (0, i, 0))],
        out_specs=pl.BlockSpec((blk, D_FEAT), lambda i: (i, 0)),
    )(partials)


# speedup vs baseline: 1.7131x; 1.0003x over previous
import functools

import jax
import jax.numpy as jnp
from jax import lax
from jax.experimental import pallas as pl
from jax.experimental.pallas import tpu as pltpu
from jax.experimental.pallas import tpu_sc as plsc

N_NODES = 10000
N_EDGES = 320000
D_FEAT = 128

NC = 2
NS = 16
LANES = 16
NW = NC * NS
CHUNK = 128
GROUPS = CHUNK // LANES
QUAD = 2
SB = 16
NSUPER = 5
EPW = NSUPER * SB * CHUNK
E_PAD = NW * EPW
RPT = 624
RTAIL = N_NODES - NS * RPT


def _scale_chunk(gbuf, val_v, c):
    def group_body(g, carry):
        v16 = val_v[c, pl.ds(g * LANES, LANES)]
        for i in range(LANES):
            s = v16.at[jnp.full((LANES,), i, jnp.int32)].get(
                mode="promise_in_bounds")
            e = g * LANES + i
            for f in range(D_FEAT // LANES):
                sl = pl.ds(f * LANES, LANES)
                gbuf[e, sl] = gbuf[e, sl] * s
        return carry

    lax.fori_loop(0, GROUPS, group_body, 0)


def _sc_body(x_hbm, row_hbm, col_hbm, val_hbm, out_hbm,
             row_v, col_v, val_v, gb0, gb1, acc_sh,
             gsem0, gsem1, ssem0, ssem1):
    gbufs = (gb0, gb1)
    gsems = (gsem0, gsem1)
    ssems = (ssem0, ssem1)
    cid = lax.axis_index("c")
    sid = lax.axis_index("s")
    wid = sid * NC + cid

    def zfill(r, carry):
        for f in range(D_FEAT // LANES):
            gb0[r, pl.ds(f * LANES, LANES)] = jnp.zeros((LANES,), jnp.float32)
        return carry

    lax.fori_loop(0, CHUNK, zfill, 0)
    ZROWS = 78
    for z in range(8):
        pltpu.sync_copy(gb0.at[pl.ds(0, ZROWS)],
                        acc_sh.at[pl.ds(sid * RPT + z * ZROWS, ZROWS)])

    @pl.when(sid == 0)
    def _():
        pltpu.sync_copy(gb0.at[pl.ds(0, RTAIL)],
                        acc_sh.at[pl.ds(NS * RPT, RTAIL)])

    plsc.subcore_barrier()

    def super_body(s_i, carry0):
        pltpu.sync_copy(row_hbm.at[wid, s_i], row_v)
        pltpu.sync_copy(col_hbm.at[wid, s_i], col_v)
        pltpu.sync_copy(val_hbm.at[wid, s_i], val_v)

        def quad_body(p, carry):
            gathers = []
            for b in range(QUAD):
                c = p * QUAD + b
                h = CHUNK // 2
                gathers.append((
                    pltpu.async_copy(x_hbm.at[col_v.at[c, pl.ds(0, h)]],
                                     gbufs[b].at[pl.ds(0, h)], gsems[b]),
                    pltpu.async_copy(x_hbm.at[col_v.at[c, pl.ds(h, h)]],
                                     gbufs[b].at[pl.ds(h, h)], gsems[b])))
            scatters = []
            for b in range(QUAD):
                c = p * QUAD + b
                gathers[b][0].wait()
                gathers[b][1].wait()
                _scale_chunk(gbufs[b], val_v, c)
                scatters.append(pltpu.async_copy(
                    gbufs[b], acc_sh.at[row_v.at[c]], ssems[b], add=True))
            for b in range(QUAD):
                scatters[b].wait()
            return carry

        lax.fori_loop(0, SB // QUAD, quad_body, 0)
        return carry0

    lax.fori_loop(0, NSUPER, super_body, 0)
    plsc.subcore_barrier()

    pltpu.sync_copy(acc_sh.at[pl.ds(sid * RPT, RPT)],
                    out_hbm.at[cid, pl.ds(sid * RPT, RPT)])

    @pl.when(sid == 0)
    def _():
        pltpu.sync_copy(acc_sh.at[pl.ds(NS * RPT, RTAIL)],
                        out_hbm.at[cid, pl.ds(NS * RPT, RTAIL)])


def _combine_body(p_ref, o_ref):
    o_ref[...] = p_ref[0] + p_ref[1]


@jax.jit
def kernel(x, adj_indices, adj_values):
    idx = adj_indices.astype(jnp.int32)
    pad = E_PAD - N_EDGES
    spread = (jnp.arange(pad, dtype=jnp.int32) * 13) % N_NODES
    row = jnp.concatenate([idx[0], spread])
    col = jnp.concatenate([idx[1], spread])
    val = jnp.concatenate([adj_values, jnp.zeros((pad,), jnp.float32)])
    row4 = row.reshape(NW, NSUPER, SB, CHUNK)
    col4 = col.reshape(NW, NSUPER, SB, CHUNK)
    val4 = val.reshape(NW, NSUPER, SB, CHUNK)

    mesh = plsc.VectorSubcoreMesh(core_axis_name="c", subcore_axis_name="s",
                                  num_cores=NC, num_subcores=NS)
    partials = pl.kernel(
        _sc_body,
        out_type=jax.ShapeDtypeStruct((NC, N_NODES, D_FEAT), jnp.float32),
        mesh=mesh,
        scratch_types=[
            pltpu.VMEM((SB, CHUNK), jnp.int32),
            pltpu.VMEM((SB, CHUNK), jnp.int32),
            pltpu.VMEM((SB, CHUNK), jnp.float32),
            pltpu.VMEM((CHUNK, D_FEAT), jnp.float32),
            pltpu.VMEM((CHUNK, D_FEAT), jnp.float32),
            pltpu.VMEM_SHARED((N_NODES, D_FEAT), jnp.float32),
            pltpu.SemaphoreType.DMA,
            pltpu.SemaphoreType.DMA,
            pltpu.SemaphoreType.DMA,
            pltpu.SemaphoreType.DMA,
        ],
    )(x, row4, col4, val4)

    blk = 2000
    return pl.pallas_call(
        _combine_body,
        out_shape=jax.ShapeDtypeStruct((N_NODES, D_FEAT), jnp.float32),
        grid=(N_NODES // blk,),
        in_specs=[pl.BlockSpec((NC, blk, D_FEAT), lambda i: (0, i, 0))],
        out_specs=pl.BlockSpec((blk, D_FEAT), lambda i: (i, 0)),
    )(partials)
